# baseline (speedup 1.0000x reference)
import jax
import jax.numpy as jnp
from jax.experimental import pallas as pl

_N = 4096
_NB = 4
_BR = 256


def _focal_body(batch_ref, pred_ref, y_ref, out_ref):
    g = pl.program_id(0)
    b_all = batch_ref[0, :]

    brow = batch_ref[0, pl.ds(g * _BR, _BR)]

    w_all = jnp.zeros((_N,), jnp.float32)
    wrow = jnp.zeros((_BR,), jnp.float32)
    for b in range(_NB):
        m = (b_all == b).astype(jnp.float32)
        inv = 1.0 / jnp.maximum(jnp.sum(m), 1.0)
        w_all = w_all + m * inv
        wrow = wrow + (brow == b).astype(jnp.float32) * inv

    x = pred_ref[...]
    y = y_ref[...].astype(jnp.float32)
    p = jax.nn.sigmoid(x)
    log_p = jax.nn.log_sigmoid(x)
    log_1mp = jax.nn.log_sigmoid(-x)
    omp = 1.0 - p
    loss = -(y * omp * omp * log_p + (1.0 - y) * p * p * log_1mp)

    eq = brow[:, None] == b_all[None, :]
    wmat = wrow[:, None] * w_all[None, :]
    contrib = jnp.sum(jnp.where(eq, loss * wmat, 0.0))

    contrib2d = contrib.reshape(1, 1)

    @pl.when(g == 0)
    def _():
        out_ref[...] = contrib2d

    @pl.when(g > 0)
    def _():
        out_ref[...] += contrib2d


def kernel(y_seg_pred, y_seg, batch):
    batch2d = batch.astype(jnp.int32).reshape(1, _N)
    total = pl.pallas_call(
        _focal_body,
        grid=(_N // _BR,),
        in_specs=[
            pl.BlockSpec((1, _N), lambda g: (0, 0)),
            pl.BlockSpec((_BR, _N), lambda g: (g, 0)),
            pl.BlockSpec((_BR, _N), lambda g: (g, 0)),
        ],
        out_specs=pl.BlockSpec((1, 1), lambda g: (0, 0)),
        out_shape=jax.ShapeDtypeStruct((1, 1), jnp.float32),
    )(batch2d, y_seg_pred, y_seg)
    batch_size = (batch[-1] + 1).astype(jnp.float32)
    return total[0, 0] / batch_size


# speedup vs baseline: 2.0612x; 2.0612x over previous
import jax
import jax.numpy as jnp
from jax.experimental import pallas as pl
from jax.experimental.pallas import tpu as pltpu

_N = 4096
_NB = 4
_T = 256
_NT = _N // _T
_G = _NT * _NT


def _focal_body(sched_ref, batch_ref, pred_ref, y_ref, out_ref, inv_ref):
    g = pl.program_id(0)
    m_valid = sched_ref[2, 0]

    @pl.when(g == 0)
    def _():
        out_ref[...] = jnp.zeros_like(out_ref)
        b_all = batch_ref[0, :]
        for b in range(_NB):
            cnt = jnp.sum((b_all == b).astype(jnp.float32))
            inv_ref[b] = 1.0 / jnp.maximum(cnt, 1.0)

    @pl.when(g < m_valid)
    def _():
        ri = sched_ref[0, g]
        ci = sched_ref[1, g]
        brow = batch_ref[0, pl.ds(ri * _T, _T)]
        bcol = batch_ref[0, pl.ds(ci * _T, _T)]
        wrow = jnp.zeros((_T,), jnp.float32)
        wcol = jnp.zeros((_T,), jnp.float32)
        for b in range(_NB):
            wrow = wrow + (brow == b).astype(jnp.float32) * inv_ref[b]
            wcol = wcol + (bcol == b).astype(jnp.float32) * inv_ref[b]

        x = pred_ref[...]
        y = y_ref[...].astype(jnp.float32)
        p = jax.nn.sigmoid(x)
        log_p = jax.nn.log_sigmoid(x)
        log_1mp = jax.nn.log_sigmoid(-x)
        omp = 1.0 - p
        loss = -(y * omp * omp * log_p + (1.0 - y) * p * p * log_1mp)

        eq = brow[:, None] == bcol[None, :]
        wmat = wrow[:, None] * wcol[None, :]
        contrib = jnp.sum(jnp.where(eq, loss * wmat, 0.0))
        out_ref[...] += contrib.reshape(1, 1)


def _make_schedule(batch):
    first = batch[:: _T]
    last = batch[_T - 1 :: _T]
    needed = (first[:, None] <= last[None, :]) & (first[None, :] <= last[:, None])
    flat = needed.reshape(-1)
    m = jnp.sum(flat.astype(jnp.int32))

    perm = jnp.argsort(~flat, stable=True).astype(jnp.int32)
    idx = jnp.where(jnp.arange(_G, dtype=jnp.int32) < m, perm, perm[m - 1])
    sched = jnp.stack(
        [idx // _NT, idx % _NT, jnp.full((_G,), m, dtype=jnp.int32)]
    )
    return sched


def kernel(y_seg_pred, y_seg, batch):
    batch = batch.astype(jnp.int32)
    sched = _make_schedule(batch)
    batch2d = batch.reshape(1, _N)
    total = pl.pallas_call(
        _focal_body,
        grid_spec=pltpu.PrefetchScalarGridSpec(
            num_scalar_prefetch=1,
            grid=(_G,),
            in_specs=[
                pl.BlockSpec((1, _N), lambda g, s: (0, 0)),
                pl.BlockSpec((_T, _T), lambda g, s: (s[0, g], s[1, g])),
                pl.BlockSpec((_T, _T), lambda g, s: (s[0, g], s[1, g])),
            ],
            out_specs=pl.BlockSpec((1, 1), lambda g, s: (0, 0)),
            scratch_shapes=[pltpu.SMEM((_NB,), jnp.float32)],
        ),
        out_shape=jax.ShapeDtypeStruct((1, 1), jnp.float32),
    )(sched, batch2d, y_seg_pred, y_seg)
    batch_size = (batch[-1] + 1).astype(jnp.float32)
    return total[0, 0] / batch_size


# speedup vs baseline: 2.2195x; 1.0768x over previous
import jax
import jax.numpy as jnp
from jax.experimental import pallas as pl
from jax.experimental.pallas import tpu as pltpu

_N = 4096
_NB = 4
_T = 256
_NT = _N // _T
_G = _NT * _NT


def _focal_body(sched_ref, batch_ref, pred_ref, y_ref, out_ref, inv_ref):
    g = pl.program_id(0)
    m_valid = sched_ref[2, 0]

    @pl.when(g == 0)
    def _():
        out_ref[...] = jnp.zeros_like(out_ref)
        b_all = batch_ref[0, :]
        for b in range(_NB):
            cnt = jnp.sum((b_all == b).astype(jnp.float32))
            inv_ref[b] = 1.0 / jnp.maximum(cnt, 1.0)

    @pl.when(g < m_valid)
    def _():
        ri = sched_ref[0, g]
        ci = sched_ref[1, g]
        brow = batch_ref[0, pl.ds(ri * _T, _T)]
        bcol = batch_ref[0, pl.ds(ci * _T, _T)]
        wrow = jnp.zeros((_T,), jnp.float32)
        wcol = jnp.zeros((_T,), jnp.float32)
        for b in range(_NB):
            wrow = wrow + (brow == b).astype(jnp.float32) * inv_ref[b]
            wcol = wcol + (bcol == b).astype(jnp.float32) * inv_ref[b]

        x = pred_ref[...]
        y = y_ref[...].astype(jnp.float32)
        log_p = jax.nn.log_sigmoid(x)
        p = jnp.exp(log_p)
        omp = 1.0 - p
        loss = -(y * omp * omp * log_p + (1.0 - y) * p * p * (log_p - x))

        eq = brow[:, None] == bcol[None, :]
        wmat = wrow[:, None] * wcol[None, :]
        contrib = jnp.sum(jnp.where(eq, loss * wmat, 0.0))
        out_ref[...] += contrib.reshape(1, 1)


def _make_schedule(batch):
    first = batch[:: _T]
    last = batch[_T - 1 :: _T]
    needed = (first[:, None] <= last[None, :]) & (first[None, :] <= last[:, None])
    flat = needed.reshape(-1)
    m = jnp.sum(flat.astype(jnp.int32))

    perm = jnp.argsort(~flat, stable=True).astype(jnp.int32)
    idx = jnp.where(jnp.arange(_G, dtype=jnp.int32) < m, perm, perm[m - 1])
    sched = jnp.stack(
        [idx // _NT, idx % _NT, jnp.full((_G,), m, dtype=jnp.int32)]
    )
    return sched


def kernel(y_seg_pred, y_seg, batch):
    batch = batch.astype(jnp.int32)
    sched = _make_schedule(batch)
    batch2d = batch.reshape(1, _N)
    total = pl.pallas_call(
        _focal_body,
        grid_spec=pltpu.PrefetchScalarGridSpec(
            num_scalar_prefetch=1,
            grid=(_G,),
            in_specs=[
                pl.BlockSpec((1, _N), lambda g, s: (0, 0)),
                pl.BlockSpec((_T, _T), lambda g, s: (s[0, g], s[1, g])),
                pl.BlockSpec((_T, _T), lambda g, s: (s[0, g], s[1, g])),
            ],
            out_specs=pl.BlockSpec((1, 1), lambda g, s: (0, 0)),
            scratch_shapes=[pltpu.SMEM((_NB,), jnp.float32)],
        ),
        out_shape=jax.ShapeDtypeStruct((1, 1), jnp.float32),
    )(sched, batch2d, y_seg_pred, y_seg)
    batch_size = (batch[-1] + 1).astype(jnp.float32)
    return total[0, 0] / batch_size
